# Initial kernel scaffold; baseline (speedup 1.0000x reference)
import jax
import jax.numpy as jnp
from jax.experimental import pallas as pl


def kernel(feat_user, feat_item, src_follows, dst_follows, W_micro_follows, a_micro_follows, W_etype_follows, src_rates, dst_rates, W_micro_rates, a_micro_rates, W_etype_rates, src_rated_by, dst_rated_by, W_micro_rated_by, a_micro_rated_by, W_etype_rated_by, src_similar, dst_similar, W_micro_similar, a_micro_similar, W_etype_similar, W_central_user, W_res_user, b_res_user, res_w_user, W_central_item, W_res_item, b_res_item, res_w_item, edge_types_attention_weight):
    raise NotImplementedError("write your pallas kernel here")


# speedup vs baseline: 8.2894x; 8.2894x over previous
import jax
import jax.numpy as jnp
from jax.experimental import pallas as pl

_BN = 1000
_H = 4
_HD = 32
_D = _H * _HD


def _mm_body(x_ref, w_ref, o_ref):
    o_ref[:, :] = jnp.dot(x_ref[:, :], w_ref[:, :],
                          preferred_element_type=jnp.float32)


def _mm(x, w):
    n, k = x.shape
    m = w.shape[1]
    return pl.pallas_call(
        _mm_body,
        grid=(n // _BN,),
        in_specs=[pl.BlockSpec((_BN, k), lambda i: (i, 0)),
                  pl.BlockSpec((k, m), lambda i: (0, 0))],
        out_specs=pl.BlockSpec((_BN, m), lambda i: (i, 0)),
        out_shape=jax.ShapeDtypeStruct((n, m), jnp.float32),
    )(x, w)


def _macro_body(cs_ref, rs1_ref, rs2_ref, r1_ref, r2_ref, res_ref, sig_ref,
                o_ref):
    cs = cs_ref[:, 0:_H]
    s1 = cs + rs1_ref[:, _H:2 * _H]
    s2 = cs + rs2_ref[:, _H:2 * _H]
    s1 = jnp.where(s1 >= 0, s1, 0.2 * s1)
    s2 = jnp.where(s2 >= 0, s2, 0.2 * s2)
    m = jnp.maximum(s1, s2)
    e1 = jnp.exp(s1 - m)
    e2 = jnp.exp(s2 - m)
    a1 = e1 / (e1 + e2)
    a2 = 1.0 - a1
    sig = sig_ref[0, 0]
    for h in range(_H):
        sl = slice(h * _HD, (h + 1) * _HD)
        mac = a1[:, h:h + 1] * r1_ref[:, sl] + a2[:, h:h + 1] * r2_ref[:, sl]
        o_ref[:, sl] = mac * sig + res_ref[:, sl] * (1.0 - sig)


def _macro(cs, rs1, rs2, r1, r2, res, sig):
    n = r1.shape[0]
    s8 = pl.BlockSpec((_BN, 2 * _H), lambda i: (i, 0))
    sD = pl.BlockSpec((_BN, _D), lambda i: (i, 0))
    s1 = pl.BlockSpec((1, 1), lambda i: (0, 0))
    return pl.pallas_call(
        _macro_body,
        grid=(n // _BN,),
        in_specs=[s8, s8, s8, sD, sD, sD, s1],
        out_specs=sD,
        out_shape=jax.ShapeDtypeStruct((n, _D), jnp.float32),
    )(cs, rs1, rs2, r1, r2, res, sig)


def _score_mat(a):
    A = jnp.zeros((_D, 2 * _H), jnp.float32)
    for h in range(_H):
        A = A.at[h * _HD:(h + 1) * _HD, h].set(a[h, :_HD])
        A = A.at[h * _HD:(h + 1) * _HD, _H + h].set(a[h, _HD:])
    return A


def _micro_edge(hs, el8, er8, src, dst, n_dst):
    e = el8[:, 0:_H][src] + er8[:, _H:2 * _H][dst]
    e = jnp.where(e >= 0, e, 0.2 * e)
    mx = jax.ops.segment_max(e, dst, num_segments=n_dst)
    mx = jnp.where(jnp.isfinite(mx), mx, 0.0)
    ex = jnp.exp(e - mx[dst])
    den = jax.ops.segment_sum(ex, dst, num_segments=n_dst)
    alpha = ex / (den[dst] + 1e-9)
    msg = hs[src] * jnp.repeat(alpha, _HD, axis=1)
    return jax.ops.segment_sum(msg, dst, num_segments=n_dst)


def kernel(feat_user, feat_item, src_follows, dst_follows, W_micro_follows, a_micro_follows, W_etype_follows, src_rates, dst_rates, W_micro_rates, a_micro_rates, W_etype_rates, src_rated_by, dst_rated_by, W_micro_rated_by, a_micro_rated_by, W_etype_rated_by, src_similar, dst_similar, W_micro_similar, a_micro_similar, W_etype_similar, W_central_user, W_res_user, b_res_user, res_w_user, W_central_item, W_res_item, b_res_item, res_w_item, edge_types_attention_weight):
    n_user = feat_user.shape[0]
    n_item = feat_item.shape[0]

    h_f = _mm(feat_user, W_micro_follows)
    s_f = _mm(h_f, _score_mat(a_micro_follows))
    agg_follows = _micro_edge(h_f, s_f, s_f, src_follows, dst_follows, n_user)

    h_ra_s = _mm(feat_user, W_micro_rates)
    h_ra_d = _mm(feat_item, W_micro_rates)
    A_ra = _score_mat(a_micro_rates)
    agg_rates = _micro_edge(h_ra_s, _mm(h_ra_s, A_ra), _mm(h_ra_d, A_ra),
                            src_rates, dst_rates, n_item)

    h_rb_s = _mm(feat_item, W_micro_rated_by)
    h_rb_d = _mm(feat_user, W_micro_rated_by)
    A_rb = _score_mat(a_micro_rated_by)
    agg_rated_by = _micro_edge(h_rb_s, _mm(h_rb_s, A_rb), _mm(h_rb_d, A_rb),
                               src_rated_by, dst_rated_by, n_user)

    h_si = _mm(feat_item, W_micro_similar)
    s_si = _mm(h_si, _score_mat(a_micro_similar))
    agg_similar = _micro_edge(h_si, s_si, s_si, src_similar, dst_similar,
                              n_item)

    B = _score_mat(edge_types_attention_weight)

    r_u1 = _mm(agg_follows, W_etype_follows)
    r_u2 = _mm(agg_rated_by, W_etype_rated_by)
    cen_u = _mm(feat_user, W_central_user)
    res_u = _mm(feat_user, W_res_user) + b_res_user[None, :]
    out_user = _macro(_mm(cen_u, B), _mm(r_u1, B), _mm(r_u2, B),
                      r_u1, r_u2, res_u,
                      jax.nn.sigmoid(res_w_user).reshape(1, 1))

    r_i1 = _mm(agg_rates, W_etype_rates)
    r_i2 = _mm(agg_similar, W_etype_similar)
    cen_i = _mm(feat_item, W_central_item)
    res_i = _mm(feat_item, W_res_item) + b_res_item[None, :]
    out_item = _macro(_mm(cen_i, B), _mm(r_i1, B), _mm(r_i2, B),
                      r_i1, r_i2, res_i,
                      jax.nn.sigmoid(res_w_item).reshape(1, 1))

    return out_user, out_item
